# baseline (speedup 1.0000x reference)
import jax
import jax.numpy as jnp
from jax import lax
from jax.experimental import pallas as pl
from jax.experimental.pallas import tpu as pltpu
from jax.experimental.pallas import tpu_sc as plsc

_N = 10000
_E = 320000
_D = 128
_NC = 2
_NS = 16
_NW = _NC * _NS
_EPT = _E // _NW
_CHUNK = 128
_EPTP = 10112
_NCHUNK = _EPTP // _CHUNK
_NPAD = 10112
_RPT = _NPAD // _NS

_mesh = plsc.VectorSubcoreMesh(core_axis_name="c", subcore_axis_name="s")


def _fill_rows(buf, nrows, value_for_lane_block):
    for l in range(8):
        v = jnp.full((16,), value_for_lane_block(l), jnp.float32)

        @pl.loop(0, nrows)
        def _(r):
            buf[r, pl.ds(16 * l, 16)] = v


def _zero_acc_slice(zbuf8, acc_sh, s):
    @pl.loop(0, _RPT // 8)
    def _(j):
        pltpu.sync_copy(zbuf8, acc_sh.at[pl.ds(s * _RPT + 8 * j, 8)])


def _deg_body(idx_hbm, out_hbm, idx_v, ones_v, zbuf_v, acc_sh):
    c = lax.axis_index("c")
    s = lax.axis_index("s")
    wid = s * _NC + c
    _fill_rows(zbuf_v, 8, lambda l: 0.0)
    _zero_acc_slice(zbuf_v, acc_sh, s)
    plsc.subcore_barrier()

    for k in (0, 1):
        _fill_rows(ones_v, _CHUNK,
                   (lambda l: 1.0 if l < 4 else 0.0) if k == 0 else
                   (lambda l: 0.0 if l < 4 else 1.0))
        pltpu.sync_copy(idx_hbm.at[k, wid], idx_v)

        @pl.loop(0, _NCHUNK)
        def _(j):
            pltpu.sync_copy(ones_v, acc_sh.at[idx_v.at[j]], add=True)

    plsc.subcore_barrier()
    pltpu.sync_copy(acc_sh.at[pl.ds(s * _RPT, _RPT)],
                    out_hbm.at[c, pl.ds(s * _RPT, _RPT)])


def _sc_degrees(idx4):
    f = pl.kernel(
        _deg_body,
        out_type=jax.ShapeDtypeStruct((_NC, _NPAD, _D), jnp.float32),
        mesh=_mesh,
        scratch_types=[
            pltpu.VMEM((_NCHUNK, _CHUNK), jnp.int32),
            pltpu.VMEM((_CHUNK, _D), jnp.float32),
            pltpu.VMEM((8, _D), jnp.float32),
            pltpu.VMEM_SHARED((_NPAD, _D), jnp.float32),
        ],
    )
    return f(idx4)


def _agg_body(h_hbm, src_hbm, dst_hbm, out_hbm,
              src_v, dst_v, buf_v, acc_sh):
    c = lax.axis_index("c")
    s = lax.axis_index("s")
    wid = s * _NC + c
    _fill_rows(buf_v, 8, lambda l: 0.0)
    _zero_acc_slice(buf_v.at[pl.ds(0, 8)], acc_sh, s)
    pltpu.sync_copy(src_hbm.at[wid], src_v)
    pltpu.sync_copy(dst_hbm.at[wid], dst_v)
    plsc.subcore_barrier()

    @pl.loop(0, _NCHUNK)
    def _(j):
        pltpu.sync_copy(h_hbm.at[src_v.at[j]], buf_v)
        pltpu.sync_copy(buf_v, acc_sh.at[dst_v.at[j]], add=True)

    plsc.subcore_barrier()
    pltpu.sync_copy(acc_sh.at[pl.ds(s * _RPT, _RPT)],
                    out_hbm.at[c, pl.ds(s * _RPT, _RPT)])


def _sc_aggregate(h, src3, dst3):
    f = pl.kernel(
        _agg_body,
        out_type=jax.ShapeDtypeStruct((_NC, _NPAD, _D), jnp.float32),
        mesh=_mesh,
        scratch_types=[
            pltpu.VMEM((_NCHUNK, _CHUNK), jnp.int32),
            pltpu.VMEM((_NCHUNK, _CHUNK), jnp.int32),
            pltpu.VMEM((_CHUNK, _D), jnp.float32),
            pltpu.VMEM_SHARED((_NPAD, _D), jnp.float32),
        ],
    )
    return f(h, src3, dst3)


_BLK = 1000
_NBLK = _N // _BLK


def _norm_from(degp_ref, k):
    deg = degp_ref[0, :, 64 * k] + degp_ref[1, :, 64 * k]
    return lax.rsqrt(jnp.maximum(deg, 1.0))


def _scale_body(feat_ref, degp_ref, o_ref):
    o_ref[...] = feat_ref[...] * _norm_from(degp_ref, 0)[:, None]


def _tc_scale_src(features, degp):
    return pl.pallas_call(
        _scale_body,
        grid=(_NBLK,),
        in_specs=[
            pl.BlockSpec((_BLK, _D), lambda i: (i, 0)),
            pl.BlockSpec((_NC, _BLK, _D), lambda i: (0, i, 0)),
        ],
        out_specs=pl.BlockSpec((_BLK, _D), lambda i: (i, 0)),
        out_shape=jax.ShapeDtypeStruct((_NPAD, _D), jnp.float32),
    )(features, degp)


def _mid_body(aggp_ref, degp_ref, w_ref, b_ref, o_ref):
    agg = aggp_ref[0] + aggp_ref[1]
    agg = agg * _norm_from(degp_ref, 1)[:, None]
    x = lax.dot_general(agg, w_ref[...], (((1,), (0,)), ((), ())),
                        precision=lax.Precision.HIGHEST,
                        preferred_element_type=jnp.float32)
    x = jnp.maximum(x + b_ref[...], 0.0)
    o_ref[...] = x * _norm_from(degp_ref, 0)[:, None]


def _tc_mid(aggp, degp, W, b):
    return pl.pallas_call(
        _mid_body,
        grid=(_NBLK,),
        in_specs=[
            pl.BlockSpec((_NC, _BLK, _D), lambda i: (0, i, 0)),
            pl.BlockSpec((_NC, _BLK, _D), lambda i: (0, i, 0)),
            pl.BlockSpec((_D, _D), lambda i: (0, 0)),
            pl.BlockSpec((1, _D), lambda i: (0, 0)),
        ],
        out_specs=pl.BlockSpec((_BLK, _D), lambda i: (i, 0)),
        out_shape=jax.ShapeDtypeStruct((_NPAD, _D), jnp.float32),
    )(aggp, degp, W, b.reshape(1, _D))


def _head_body(aggp_ref, degp_ref, w_ref, b_ref, wc_ref, bc_ref,
               o_ref, acc_ref):
    i = pl.program_id(0)
    agg = aggp_ref[0] + aggp_ref[1]
    agg = agg * _norm_from(degp_ref, 1)[:, None]
    x = lax.dot_general(agg, w_ref[...], (((1,), (0,)), ((), ())),
                        precision=lax.Precision.HIGHEST,
                        preferred_element_type=jnp.float32)
    x = jnp.maximum(x + b_ref[...], 0.0)
    part = jnp.sum(x, axis=0, keepdims=True)

    @pl.when(i == 0)
    def _():
        acc_ref[...] = jnp.zeros_like(acc_ref)

    acc_ref[0:1, :] += part

    @pl.when(i == _NBLK - 1)
    def _():
        hg = acc_ref[0:1, :] * (1.0 / _N)
        o_ref[...] = lax.dot_general(
            hg, wc_ref[...], (((1,), (0,)), ((), ())),
            precision=lax.Precision.HIGHEST,
            preferred_element_type=jnp.float32) + bc_ref[...]


def _tc_head(aggp, degp, W, b, Wc, bc):
    return pl.pallas_call(
        _head_body,
        grid=(_NBLK,),
        in_specs=[
            pl.BlockSpec((_NC, _BLK, _D), lambda i: (0, i, 0)),
            pl.BlockSpec((_NC, _BLK, _D), lambda i: (0, i, 0)),
            pl.BlockSpec((_D, _D), lambda i: (0, 0)),
            pl.BlockSpec((1, _D), lambda i: (0, 0)),
            pl.BlockSpec((_D, 10), lambda i: (0, 0)),
            pl.BlockSpec((1, 10), lambda i: (0, 0)),
        ],
        out_specs=pl.BlockSpec((1, 10), lambda i: (0, 0)),
        out_shape=jax.ShapeDtypeStruct((1, 10), jnp.float32),
        scratch_shapes=[pltpu.VMEM((8, _D), jnp.float32)],
    )(aggp, degp, W, b.reshape(1, _D), Wc, bc.reshape(1, 10))


def kernel(features, edge_index, W1, b1, W2, b2, Wc, bc):
    idx3 = edge_index.reshape(2, _NW, _EPT)
    idx3 = jnp.pad(idx3, ((0, 0), (0, 0), (0, _EPTP - _EPT)),
                   constant_values=_NPAD - 1)
    idx4 = idx3.reshape(2, _NW, _NCHUNK, _CHUNK)
    src3 = idx4[0]
    dst3 = idx4[1]

    degp = _sc_degrees(idx4)
    h1 = _tc_scale_src(features, degp)
    agg1 = _sc_aggregate(h1, src3, dst3)
    h2 = _tc_mid(agg1, degp, W1, b1)
    agg2 = _sc_aggregate(h2, src3, dst3)
    return _tc_head(agg2, degp, W2, b2, Wc, bc)


# speedup vs baseline: 1.0703x; 1.0703x over previous
import jax
import jax.numpy as jnp
from jax import lax
from jax.experimental import pallas as pl
from jax.experimental.pallas import tpu as pltpu
from jax.experimental.pallas import tpu_sc as plsc

_N = 10000
_E = 320000
_D = 128
_NC = 2
_NS = 16
_NW = _NC * _NS
_EPT = _E // _NW
_CHUNK = 64
_EPTP = 10048
_NCHUNK = _EPTP // _CHUNK
_NPAD = 10112
_RPT = _NPAD // _NS

_mesh = plsc.VectorSubcoreMesh(core_axis_name="c", subcore_axis_name="s")


def _fill_rows(buf, nrows, value_for_lane_block):
    for l in range(8):
        v = jnp.full((16,), value_for_lane_block(l), jnp.float32)

        @pl.loop(0, nrows)
        def _(r):
            buf[r, pl.ds(16 * l, 16)] = v


def _zero_acc_slice(zbuf8, acc_sh, s):
    @pl.loop(0, _RPT // 8)
    def _(j):
        pltpu.sync_copy(zbuf8, acc_sh.at[pl.ds(s * _RPT + 8 * j, 8)])


def _deg_body(idx_hbm, out_hbm, idx_v, ones_v, zbuf_v, acc_sh):
    c = lax.axis_index("c")
    s = lax.axis_index("s")
    wid = s * _NC + c
    _fill_rows(zbuf_v, 8, lambda l: 0.0)
    _zero_acc_slice(zbuf_v, acc_sh, s)
    plsc.subcore_barrier()

    for k in (0, 1):
        _fill_rows(ones_v, _CHUNK,
                   (lambda l: 1.0 if l < 4 else 0.0) if k == 0 else
                   (lambda l: 0.0 if l < 4 else 1.0))
        pltpu.sync_copy(idx_hbm.at[k, wid], idx_v)

        @pl.loop(0, _NCHUNK)
        def _(j):
            pltpu.sync_copy(ones_v, acc_sh.at[idx_v.at[j]], add=True)

    plsc.subcore_barrier()
    pltpu.sync_copy(acc_sh.at[pl.ds(s * _RPT, _RPT)],
                    out_hbm.at[c, pl.ds(s * _RPT, _RPT)])


def _sc_degrees(idx4):
    f = pl.kernel(
        _deg_body,
        out_type=jax.ShapeDtypeStruct((_NC, _NPAD, _D), jnp.float32),
        mesh=_mesh,
        scratch_types=[
            pltpu.VMEM((_NCHUNK, _CHUNK), jnp.int32),
            pltpu.VMEM((_CHUNK, _D), jnp.float32),
            pltpu.VMEM((8, _D), jnp.float32),
            pltpu.VMEM_SHARED((_NPAD, _D), jnp.float32),
        ],
    )
    return f(idx4)


def _agg_body(h_hbm, src_hbm, dst_hbm, out_hbm,
              src_v, dst_v, buf_v, acc_sh):
    c = lax.axis_index("c")
    s = lax.axis_index("s")
    wid = s * _NC + c
    _fill_rows(buf_v, 8, lambda l: 0.0)
    _zero_acc_slice(buf_v.at[pl.ds(0, 8)], acc_sh, s)
    pltpu.sync_copy(src_hbm.at[wid], src_v)
    pltpu.sync_copy(dst_hbm.at[wid], dst_v)
    plsc.subcore_barrier()

    @pl.loop(0, _NCHUNK)
    def _(j):
        pltpu.sync_copy(h_hbm.at[src_v.at[j]], buf_v)
        pltpu.sync_copy(buf_v, acc_sh.at[dst_v.at[j]], add=True)

    plsc.subcore_barrier()
    pltpu.sync_copy(acc_sh.at[pl.ds(s * _RPT, _RPT)],
                    out_hbm.at[c, pl.ds(s * _RPT, _RPT)])


def _sc_aggregate(h, src3, dst3):
    f = pl.kernel(
        _agg_body,
        out_type=jax.ShapeDtypeStruct((_NC, _NPAD, _D), jnp.float32),
        mesh=_mesh,
        scratch_types=[
            pltpu.VMEM((_NCHUNK, _CHUNK), jnp.int32),
            pltpu.VMEM((_NCHUNK, _CHUNK), jnp.int32),
            pltpu.VMEM((_CHUNK, _D), jnp.float32),
            pltpu.VMEM_SHARED((_NPAD, _D), jnp.float32),
        ],
    )
    return f(h, src3, dst3)


_BLK = 1000
_NBLK = _N // _BLK


def _norm_from(degp_ref, k):
    deg = degp_ref[0, :, 64 * k] + degp_ref[1, :, 64 * k]
    return lax.rsqrt(jnp.maximum(deg, 1.0))


def _scale_body(feat_ref, degp_ref, o_ref):
    o_ref[...] = feat_ref[...] * _norm_from(degp_ref, 0)[:, None]


def _tc_scale_src(features, degp):
    return pl.pallas_call(
        _scale_body,
        grid=(_NBLK,),
        in_specs=[
            pl.BlockSpec((_BLK, _D), lambda i: (i, 0)),
            pl.BlockSpec((_NC, _BLK, _D), lambda i: (0, i, 0)),
        ],
        out_specs=pl.BlockSpec((_BLK, _D), lambda i: (i, 0)),
        out_shape=jax.ShapeDtypeStruct((_NPAD, _D), jnp.float32),
    )(features, degp)


def _mid_body(aggp_ref, degp_ref, w_ref, b_ref, o_ref):
    agg = aggp_ref[0] + aggp_ref[1]
    agg = agg * _norm_from(degp_ref, 1)[:, None]
    x = lax.dot_general(agg, w_ref[...], (((1,), (0,)), ((), ())),
                        precision=lax.Precision.HIGHEST,
                        preferred_element_type=jnp.float32)
    x = jnp.maximum(x + b_ref[...], 0.0)
    o_ref[...] = x * _norm_from(degp_ref, 0)[:, None]


def _tc_mid(aggp, degp, W, b):
    return pl.pallas_call(
        _mid_body,
        grid=(_NBLK,),
        in_specs=[
            pl.BlockSpec((_NC, _BLK, _D), lambda i: (0, i, 0)),
            pl.BlockSpec((_NC, _BLK, _D), lambda i: (0, i, 0)),
            pl.BlockSpec((_D, _D), lambda i: (0, 0)),
            pl.BlockSpec((1, _D), lambda i: (0, 0)),
        ],
        out_specs=pl.BlockSpec((_BLK, _D), lambda i: (i, 0)),
        out_shape=jax.ShapeDtypeStruct((_NPAD, _D), jnp.float32),
    )(aggp, degp, W, b.reshape(1, _D))


def _head_body(aggp_ref, degp_ref, w_ref, b_ref, wc_ref, bc_ref,
               o_ref, acc_ref):
    i = pl.program_id(0)
    agg = aggp_ref[0] + aggp_ref[1]
    agg = agg * _norm_from(degp_ref, 1)[:, None]
    x = lax.dot_general(agg, w_ref[...], (((1,), (0,)), ((), ())),
                        precision=lax.Precision.HIGHEST,
                        preferred_element_type=jnp.float32)
    x = jnp.maximum(x + b_ref[...], 0.0)
    part = jnp.sum(x, axis=0, keepdims=True)

    @pl.when(i == 0)
    def _():
        acc_ref[...] = jnp.zeros_like(acc_ref)

    acc_ref[0:1, :] += part

    @pl.when(i == _NBLK - 1)
    def _():
        hg = acc_ref[0:1, :] * (1.0 / _N)
        o_ref[...] = lax.dot_general(
            hg, wc_ref[...], (((1,), (0,)), ((), ())),
            precision=lax.Precision.HIGHEST,
            preferred_element_type=jnp.float32) + bc_ref[...]


def _tc_head(aggp, degp, W, b, Wc, bc):
    return pl.pallas_call(
        _head_body,
        grid=(_NBLK,),
        in_specs=[
            pl.BlockSpec((_NC, _BLK, _D), lambda i: (0, i, 0)),
            pl.BlockSpec((_NC, _BLK, _D), lambda i: (0, i, 0)),
            pl.BlockSpec((_D, _D), lambda i: (0, 0)),
            pl.BlockSpec((1, _D), lambda i: (0, 0)),
            pl.BlockSpec((_D, 10), lambda i: (0, 0)),
            pl.BlockSpec((1, 10), lambda i: (0, 0)),
        ],
        out_specs=pl.BlockSpec((1, 10), lambda i: (0, 0)),
        out_shape=jax.ShapeDtypeStruct((1, 10), jnp.float32),
        scratch_shapes=[pltpu.VMEM((8, _D), jnp.float32)],
    )(aggp, degp, W, b.reshape(1, _D), Wc, bc.reshape(1, 10))


def kernel(features, edge_index, W1, b1, W2, b2, Wc, bc):
    idx3 = edge_index.reshape(2, _NW, _EPT)
    idx3 = jnp.pad(idx3, ((0, 0), (0, 0), (0, _EPTP - _EPT)),
                   constant_values=_NPAD - 1)
    idx4 = idx3.reshape(2, _NW, _NCHUNK, _CHUNK)
    src3 = idx4[0]
    dst3 = idx4[1]

    degp = _sc_degrees(idx4)
    h1 = _tc_scale_src(features, degp)
    agg1 = _sc_aggregate(h1, src3, dst3)
    h2 = _tc_mid(agg1, degp, W1, b1)
    agg2 = _sc_aggregate(h2, src3, dst3)
    return _tc_head(agg2, degp, W2, b2, Wc, bc)


# speedup vs baseline: 1.3178x; 1.2312x over previous
import jax
import jax.numpy as jnp
from jax import lax
from jax.experimental import pallas as pl
from jax.experimental.pallas import tpu as pltpu
from jax.experimental.pallas import tpu_sc as plsc

_N = 10000
_E = 320000
_D = 128
_NC = 2
_NS = 16
_NW = _NC * _NS
_EPT = _E // _NW
_CHUNK = 80
_EPTP = 10000
_NCHUNK = _EPTP // _CHUNK
_NPAD = 10112
_RPT = _NPAD // _NS

_mesh = plsc.VectorSubcoreMesh(core_axis_name="c", subcore_axis_name="s")


def _fill_rows(buf, nrows, value_for_lane_block):
    for l in range(8):
        v = jnp.full((16,), value_for_lane_block(l), jnp.float32)

        @pl.loop(0, nrows)
        def _(r):
            buf[r, pl.ds(16 * l, 16)] = v


def _zero_acc_slice(zbuf8, acc_sh, s):
    @pl.loop(0, _RPT // 8)
    def _(j):
        pltpu.sync_copy(zbuf8, acc_sh.at[pl.ds(s * _RPT + 8 * j, 8)])


def _deg_body(idx_hbm, out_hbm, idx_v, ones_v, zbuf_v, acc_sh):
    c = lax.axis_index("c")
    s = lax.axis_index("s")
    wid = s * _NC + c
    _fill_rows(zbuf_v, 8, lambda l: 0.0)
    _zero_acc_slice(zbuf_v, acc_sh, s)
    plsc.subcore_barrier()

    for k in (0, 1):
        _fill_rows(ones_v, _CHUNK,
                   (lambda l: 1.0 if l < 4 else 0.0) if k == 0 else
                   (lambda l: 0.0 if l < 4 else 1.0))
        pltpu.sync_copy(idx_hbm.at[k, wid], idx_v)

        @pl.loop(0, _NCHUNK)
        def _(j):
            pltpu.sync_copy(ones_v, acc_sh.at[idx_v.at[j]], add=True)

    plsc.subcore_barrier()
    pltpu.sync_copy(acc_sh.at[pl.ds(s * _RPT, _RPT)],
                    out_hbm.at[c, pl.ds(s * _RPT, _RPT)])


def _sc_degrees(idx4):
    f = pl.kernel(
        _deg_body,
        out_type=jax.ShapeDtypeStruct((_NC, _NPAD, _D), jnp.float32),
        mesh=_mesh,
        scratch_types=[
            pltpu.VMEM((_NCHUNK, _CHUNK), jnp.int32),
            pltpu.VMEM((_CHUNK, _D), jnp.float32),
            pltpu.VMEM((8, _D), jnp.float32),
            pltpu.VMEM_SHARED((_NPAD, _D), jnp.float32),
        ],
    )
    return f(idx4)


def _agg_body(h_hbm, src_hbm, dst_hbm, out_hbm,
              src_v, dst_v, buf_v, acc_sh):
    c = lax.axis_index("c")
    s = lax.axis_index("s")
    wid = s * _NC + c
    _fill_rows(buf_v, 8, lambda l: 0.0)
    _zero_acc_slice(buf_v.at[pl.ds(0, 8)], acc_sh, s)
    pltpu.sync_copy(src_hbm.at[wid], src_v)
    pltpu.sync_copy(dst_hbm.at[wid], dst_v)
    plsc.subcore_barrier()

    @pl.loop(0, _NCHUNK)
    def _(j):
        pltpu.sync_copy(h_hbm.at[src_v.at[j]], buf_v)
        pltpu.sync_copy(buf_v, acc_sh.at[dst_v.at[j]], add=True)

    plsc.subcore_barrier()
    pltpu.sync_copy(acc_sh.at[pl.ds(s * _RPT, _RPT)],
                    out_hbm.at[c, pl.ds(s * _RPT, _RPT)])


def _sc_aggregate(h, src3, dst3):
    f = pl.kernel(
        _agg_body,
        out_type=jax.ShapeDtypeStruct((_NC, _NPAD, _D), jnp.float32),
        mesh=_mesh,
        scratch_types=[
            pltpu.VMEM((_NCHUNK, _CHUNK), jnp.int32),
            pltpu.VMEM((_NCHUNK, _CHUNK), jnp.int32),
            pltpu.VMEM((_CHUNK, _D), jnp.float32),
            pltpu.VMEM_SHARED((_NPAD, _D), jnp.float32),
        ],
    )
    return f(h, src3, dst3)


_BLK = 1000
_NBLK = _N // _BLK


def _norm_from(degp_ref, k):
    deg = degp_ref[0, :, 64 * k] + degp_ref[1, :, 64 * k]
    return lax.rsqrt(jnp.maximum(deg, 1.0))


def _scale_body(feat_ref, degp_ref, o_ref):
    o_ref[...] = feat_ref[...] * _norm_from(degp_ref, 0)[:, None]


def _tc_scale_src(features, degp):
    return pl.pallas_call(
        _scale_body,
        grid=(_NBLK,),
        in_specs=[
            pl.BlockSpec((_BLK, _D), lambda i: (i, 0)),
            pl.BlockSpec((_NC, _BLK, _D), lambda i: (0, i, 0)),
        ],
        out_specs=pl.BlockSpec((_BLK, _D), lambda i: (i, 0)),
        out_shape=jax.ShapeDtypeStruct((_NPAD, _D), jnp.float32),
    )(features, degp)


def _mid_body(aggp_ref, degp_ref, w_ref, b_ref, o_ref):
    agg = aggp_ref[0] + aggp_ref[1]
    agg = agg * _norm_from(degp_ref, 1)[:, None]
    x = lax.dot_general(agg, w_ref[...], (((1,), (0,)), ((), ())),
                        precision=lax.Precision.HIGHEST,
                        preferred_element_type=jnp.float32)
    x = jnp.maximum(x + b_ref[...], 0.0)
    o_ref[...] = x * _norm_from(degp_ref, 0)[:, None]


def _tc_mid(aggp, degp, W, b):
    return pl.pallas_call(
        _mid_body,
        grid=(_NBLK,),
        in_specs=[
            pl.BlockSpec((_NC, _BLK, _D), lambda i: (0, i, 0)),
            pl.BlockSpec((_NC, _BLK, _D), lambda i: (0, i, 0)),
            pl.BlockSpec((_D, _D), lambda i: (0, 0)),
            pl.BlockSpec((1, _D), lambda i: (0, 0)),
        ],
        out_specs=pl.BlockSpec((_BLK, _D), lambda i: (i, 0)),
        out_shape=jax.ShapeDtypeStruct((_NPAD, _D), jnp.float32),
    )(aggp, degp, W, b.reshape(1, _D))


def _head_body(aggp_ref, degp_ref, w_ref, b_ref, wc_ref, bc_ref,
               o_ref, acc_ref):
    i = pl.program_id(0)
    agg = aggp_ref[0] + aggp_ref[1]
    agg = agg * _norm_from(degp_ref, 1)[:, None]
    x = lax.dot_general(agg, w_ref[...], (((1,), (0,)), ((), ())),
                        precision=lax.Precision.HIGHEST,
                        preferred_element_type=jnp.float32)
    x = jnp.maximum(x + b_ref[...], 0.0)
    part = jnp.sum(x, axis=0, keepdims=True)

    @pl.when(i == 0)
    def _():
        acc_ref[...] = jnp.zeros_like(acc_ref)

    acc_ref[0:1, :] += part

    @pl.when(i == _NBLK - 1)
    def _():
        hg = acc_ref[0:1, :] * (1.0 / _N)
        o_ref[...] = lax.dot_general(
            hg, wc_ref[...], (((1,), (0,)), ((), ())),
            precision=lax.Precision.HIGHEST,
            preferred_element_type=jnp.float32) + bc_ref[...]


def _tc_head(aggp, degp, W, b, Wc, bc):
    return pl.pallas_call(
        _head_body,
        grid=(_NBLK,),
        in_specs=[
            pl.BlockSpec((_NC, _BLK, _D), lambda i: (0, i, 0)),
            pl.BlockSpec((_NC, _BLK, _D), lambda i: (0, i, 0)),
            pl.BlockSpec((_D, _D), lambda i: (0, 0)),
            pl.BlockSpec((1, _D), lambda i: (0, 0)),
            pl.BlockSpec((_D, 10), lambda i: (0, 0)),
            pl.BlockSpec((1, 10), lambda i: (0, 0)),
        ],
        out_specs=pl.BlockSpec((1, 10), lambda i: (0, 0)),
        out_shape=jax.ShapeDtypeStruct((1, 10), jnp.float32),
        scratch_shapes=[pltpu.VMEM((8, _D), jnp.float32)],
    )(aggp, degp, W, b.reshape(1, _D), Wc, bc.reshape(1, 10))


def kernel(features, edge_index, W1, b1, W2, b2, Wc, bc):
    idx3 = edge_index.reshape(2, _NW, _EPT)
    idx3 = jnp.pad(idx3, ((0, 0), (0, 0), (0, _EPTP - _EPT)),
                   constant_values=_NPAD - 1)
    idx4 = idx3.reshape(2, _NW, _NCHUNK, _CHUNK)
    src3 = idx4[0]
    dst3 = idx4[1]

    degp = _sc_degrees(idx4)
    h1 = _tc_scale_src(features, degp)
    agg1 = _sc_aggregate(h1, src3, dst3)
    h2 = _tc_mid(agg1, degp, W1, b1)
    agg2 = _sc_aggregate(h2, src3, dst3)
    return _tc_head(agg2, degp, W2, b2, Wc, bc)


# speedup vs baseline: 1.5927x; 1.2086x over previous
import jax
import jax.numpy as jnp
from jax import lax
from jax.experimental import pallas as pl
from jax.experimental.pallas import tpu as pltpu
from jax.experimental.pallas import tpu_sc as plsc

_N = 10000
_E = 320000
_D = 128
_NC = 2
_NS = 16
_NW = _NC * _NS
_EPT = _E // _NW
_CHUNK = 80
_EPTP = 10000
_NCHUNK = _EPTP // _CHUNK
_NPAD = 10112
_RPT = _NPAD // _NS

_NHIST = 10240
_SPT = _NHIST // _NS

_mesh = plsc.VectorSubcoreMesh(core_axis_name="c", subcore_axis_name="s")

import dataclasses as _dc
_cp = pltpu.CompilerParams()
if "needs_layout_passes" in pltpu.CompilerParams.__dataclass_fields__:
    _cp = _dc.replace(_cp, needs_layout_passes=False)


def _fill_rows(buf, nrows, value_for_lane_block):
    for l in range(8):
        v = jnp.full((16,), value_for_lane_block(l), jnp.float32)

        @pl.loop(0, nrows)
        def _(r):
            buf[r, pl.ds(16 * l, 16)] = v


def _zero_acc_slice(zbuf8, acc_sh, s):
    @pl.loop(0, _RPT // 8)
    def _(j):
        pltpu.sync_copy(zbuf8, acc_sh.at[pl.ds(s * _RPT + 8 * j, 8)])


def _deg_body(idx_hbm, out_hbm, idx_v, ho_v, hi_v, mrg_v, row_v, stage_sh):
    c = lax.axis_index("c")
    s = lax.axis_index("s")
    wid = s * _NC + c
    z16 = jnp.zeros((16,), jnp.float32)
    ones16 = jnp.full((16,), 1.0, jnp.float32)

    @pl.loop(0, _NHIST // 16)
    def _(t):
        ho_v[pl.ds(16 * t, 16)] = z16
        hi_v[pl.ds(16 * t, 16)] = z16

    for k, hv in ((0, ho_v), (1, hi_v)):
        pltpu.sync_copy(idx_hbm.at[k, wid], idx_v)

        @pl.loop(0, _EPT // 16)
        def _(t):
            iv = idx_v[pl.ds(16 * t, 16)]
            plsc.addupdate_scatter(hv, [iv], ones16)

    pltpu.sync_copy(ho_v, stage_sh.at[s, 0])
    pltpu.sync_copy(hi_v, stage_sh.at[s, 1])
    plsc.subcore_barrier()

    for k, hv in ((0, ho_v), (1, hi_v)):
        pltpu.sync_copy(stage_sh.at[:, k, pl.ds(s * _SPT, _SPT)], mrg_v)

        @pl.loop(0, _SPT // 16)
        def _(u):
            acc = mrg_v[0, pl.ds(16 * u, 16)]
            for r in range(1, _NS):
                acc = acc + mrg_v[r, pl.ds(16 * u, 16)]
            hv[pl.ds(16 * u, 16)] = acc

    @pl.loop(0, _SPT // 64)
    def _(g):
        @pl.loop(0, 4)
        def _(q):
            vo = ho_v[pl.ds(64 * g + 16 * q, 16)]
            vi = hi_v[pl.ds(64 * g + 16 * q, 16)]
            for e in range(16):
                r = 16 * q + e
                row_v[r, pl.ds(0, 16)] = jnp.full((16,), 1.0,
                                                  jnp.float32) * vo[e]
                row_v[r, pl.ds(64, 16)] = jnp.full((16,), 1.0,
                                                   jnp.float32) * vi[e]

        pltpu.sync_copy(row_v,
                        out_hbm.at[c, pl.ds(s * _SPT + 64 * g, 64)])


def _sc_degrees(idx3):
    f = pl.kernel(
        _deg_body,
        out_type=jax.ShapeDtypeStruct((_NC, _NHIST, _D), jnp.float32),
        mesh=_mesh,
        compiler_params=_cp,
        scratch_types=[
            pltpu.VMEM((_EPT,), jnp.int32),
            pltpu.VMEM((_NHIST,), jnp.float32),
            pltpu.VMEM((_NHIST,), jnp.float32),
            pltpu.VMEM((_NS, _SPT), jnp.float32),
            pltpu.VMEM((64, _D), jnp.float32),
            pltpu.VMEM_SHARED((_NS, 2, _NHIST), jnp.float32),
        ],
    )
    return f(idx3)


def _agg_body(h_hbm, src_hbm, dst_hbm, out_hbm,
              src_v, dst_v, buf_v, acc_sh):
    c = lax.axis_index("c")
    s = lax.axis_index("s")
    wid = s * _NC + c
    _fill_rows(buf_v, 8, lambda l: 0.0)
    _zero_acc_slice(buf_v.at[pl.ds(0, 8)], acc_sh, s)
    pltpu.sync_copy(src_hbm.at[wid], src_v)
    pltpu.sync_copy(dst_hbm.at[wid], dst_v)
    plsc.subcore_barrier()

    @pl.loop(0, _NCHUNK)
    def _(j):
        pltpu.sync_copy(h_hbm.at[src_v.at[j]], buf_v)
        pltpu.sync_copy(buf_v, acc_sh.at[dst_v.at[j]], add=True)

    plsc.subcore_barrier()
    pltpu.sync_copy(acc_sh.at[pl.ds(s * _RPT, _RPT)],
                    out_hbm.at[c, pl.ds(s * _RPT, _RPT)])


def _sc_aggregate(h, src3, dst3):
    f = pl.kernel(
        _agg_body,
        out_type=jax.ShapeDtypeStruct((_NC, _NPAD, _D), jnp.float32),
        mesh=_mesh,
        scratch_types=[
            pltpu.VMEM((_NCHUNK, _CHUNK), jnp.int32),
            pltpu.VMEM((_NCHUNK, _CHUNK), jnp.int32),
            pltpu.VMEM((_CHUNK, _D), jnp.float32),
            pltpu.VMEM_SHARED((_NPAD, _D), jnp.float32),
        ],
    )
    return f(h, src3, dst3)


_BLK = 1000
_NBLK = _N // _BLK


def _norm_from(degp_ref, k):
    deg = degp_ref[0, :, 64 * k] + degp_ref[1, :, 64 * k]
    return lax.rsqrt(jnp.maximum(deg, 1.0))


def _scale_body(feat_ref, degp_ref, o_ref):
    o_ref[...] = feat_ref[...] * _norm_from(degp_ref, 0)[:, None]


def _tc_scale_src(features, degp):
    return pl.pallas_call(
        _scale_body,
        grid=(_NBLK,),
        in_specs=[
            pl.BlockSpec((_BLK, _D), lambda i: (i, 0)),
            pl.BlockSpec((_NC, _BLK, _D), lambda i: (0, i, 0)),
        ],
        out_specs=pl.BlockSpec((_BLK, _D), lambda i: (i, 0)),
        out_shape=jax.ShapeDtypeStruct((_NPAD, _D), jnp.float32),
    )(features, degp)


def _mid_body(aggp_ref, degp_ref, w_ref, b_ref, o_ref):
    agg = aggp_ref[0] + aggp_ref[1]
    agg = agg * _norm_from(degp_ref, 1)[:, None]
    x = lax.dot_general(agg, w_ref[...], (((1,), (0,)), ((), ())),
                        precision=lax.Precision.HIGHEST,
                        preferred_element_type=jnp.float32)
    x = jnp.maximum(x + b_ref[...], 0.0)
    o_ref[...] = x * _norm_from(degp_ref, 0)[:, None]


def _tc_mid(aggp, degp, W, b):
    return pl.pallas_call(
        _mid_body,
        grid=(_NBLK,),
        in_specs=[
            pl.BlockSpec((_NC, _BLK, _D), lambda i: (0, i, 0)),
            pl.BlockSpec((_NC, _BLK, _D), lambda i: (0, i, 0)),
            pl.BlockSpec((_D, _D), lambda i: (0, 0)),
            pl.BlockSpec((1, _D), lambda i: (0, 0)),
        ],
        out_specs=pl.BlockSpec((_BLK, _D), lambda i: (i, 0)),
        out_shape=jax.ShapeDtypeStruct((_NPAD, _D), jnp.float32),
    )(aggp, degp, W, b.reshape(1, _D))


def _head_body(aggp_ref, degp_ref, w_ref, b_ref, wc_ref, bc_ref,
               o_ref, acc_ref):
    i = pl.program_id(0)
    agg = aggp_ref[0] + aggp_ref[1]
    agg = agg * _norm_from(degp_ref, 1)[:, None]
    x = lax.dot_general(agg, w_ref[...], (((1,), (0,)), ((), ())),
                        precision=lax.Precision.HIGHEST,
                        preferred_element_type=jnp.float32)
    x = jnp.maximum(x + b_ref[...], 0.0)
    part = jnp.sum(x, axis=0, keepdims=True)

    @pl.when(i == 0)
    def _():
        acc_ref[...] = jnp.zeros_like(acc_ref)

    acc_ref[0:1, :] += part

    @pl.when(i == _NBLK - 1)
    def _():
        hg = acc_ref[0:1, :] * (1.0 / _N)
        o_ref[...] = lax.dot_general(
            hg, wc_ref[...], (((1,), (0,)), ((), ())),
            precision=lax.Precision.HIGHEST,
            preferred_element_type=jnp.float32) + bc_ref[...]


def _tc_head(aggp, degp, W, b, Wc, bc):
    return pl.pallas_call(
        _head_body,
        grid=(_NBLK,),
        in_specs=[
            pl.BlockSpec((_NC, _BLK, _D), lambda i: (0, i, 0)),
            pl.BlockSpec((_NC, _BLK, _D), lambda i: (0, i, 0)),
            pl.BlockSpec((_D, _D), lambda i: (0, 0)),
            pl.BlockSpec((1, _D), lambda i: (0, 0)),
            pl.BlockSpec((_D, 10), lambda i: (0, 0)),
            pl.BlockSpec((1, 10), lambda i: (0, 0)),
        ],
        out_specs=pl.BlockSpec((1, 10), lambda i: (0, 0)),
        out_shape=jax.ShapeDtypeStruct((1, 10), jnp.float32),
        scratch_shapes=[pltpu.VMEM((8, _D), jnp.float32)],
    )(aggp, degp, W, b.reshape(1, _D), Wc, bc.reshape(1, 10))


def kernel(features, edge_index, W1, b1, W2, b2, Wc, bc):
    idx3 = edge_index.reshape(2, _NW, _EPT)
    idxp = jnp.pad(idx3, ((0, 0), (0, 0), (0, _EPTP - _EPT)),
                   constant_values=_NPAD - 1)
    idx4 = idxp.reshape(2, _NW, _NCHUNK, _CHUNK)
    src3 = idx4[0]
    dst3 = idx4[1]

    degp = _sc_degrees(idx3)
    h1 = _tc_scale_src(features, degp)
    agg1 = _sc_aggregate(h1, src3, dst3)
    h2 = _tc_mid(agg1, degp, W1, b1)
    agg2 = _sc_aggregate(h2, src3, dst3)
    return _tc_head(agg2, degp, W2, b2, Wc, bc)


# speedup vs baseline: 1.9778x; 1.2418x over previous
import jax
import jax.numpy as jnp
from jax import lax
from jax.experimental import pallas as pl
from jax.experimental.pallas import tpu as pltpu
from jax.experimental.pallas import tpu_sc as plsc

_N = 10000
_E = 320000
_D = 128
_NC = 2
_NS = 16
_NW = _NC * _NS
_EPT = _E // _NW
_CHUNK = 80
_EPTP = 10000
_NCHUNK = _EPTP // _CHUNK
_NPAD = 10112
_RPT = _NPAD // _NS

_NHIST = 10240
_SPT = _NHIST // _NS

_mesh = plsc.VectorSubcoreMesh(core_axis_name="c", subcore_axis_name="s")

import dataclasses as _dc
_cp = pltpu.CompilerParams()
if "needs_layout_passes" in pltpu.CompilerParams.__dataclass_fields__:
    _cp = _dc.replace(_cp, needs_layout_passes=False)


def _fill_rows(buf, nrows, value_for_lane_block):
    for l in range(8):
        v = jnp.full((16,), value_for_lane_block(l), jnp.float32)

        @pl.loop(0, nrows)
        def _(r):
            buf[r, pl.ds(16 * l, 16)] = v


def _zero_acc_slice(zbuf8, acc_sh, s):
    @pl.loop(0, _RPT // 8)
    def _(j):
        pltpu.sync_copy(zbuf8, acc_sh.at[pl.ds(s * _RPT + 8 * j, 8)])


def _deg_body(idx_hbm, out_hbm, idx_v, ho_v, hi_v, mrg_v, row_v, stage_sh):
    c = lax.axis_index("c")
    s = lax.axis_index("s")
    wid = s * _NC + c
    z16 = jnp.zeros((16,), jnp.float32)
    ones16 = jnp.full((16,), 1.0, jnp.float32)

    @pl.loop(0, _NHIST // 16)
    def _(t):
        ho_v[pl.ds(16 * t, 16)] = z16
        hi_v[pl.ds(16 * t, 16)] = z16

    for k, hv in ((0, ho_v), (1, hi_v)):
        pltpu.sync_copy(idx_hbm.at[k, wid], idx_v)

        @pl.loop(0, _EPT // 16)
        def _(t):
            iv = idx_v[pl.ds(16 * t, 16)]
            plsc.addupdate_scatter(hv, [iv], ones16)

    pltpu.sync_copy(ho_v, stage_sh.at[s, 0])
    pltpu.sync_copy(hi_v, stage_sh.at[s, 1])
    plsc.subcore_barrier()

    for k, hv in ((0, ho_v), (1, hi_v)):
        pltpu.sync_copy(stage_sh.at[:, k, pl.ds(s * _SPT, _SPT)], mrg_v)

        @pl.loop(0, _SPT // 16)
        def _(u):
            acc = mrg_v[0, pl.ds(16 * u, 16)]
            for r in range(1, _NS):
                acc = acc + mrg_v[r, pl.ds(16 * u, 16)]
            hv[pl.ds(16 * u, 16)] = acc

    @pl.loop(0, _SPT // 64)
    def _(g):
        @pl.loop(0, 4)
        def _(q):
            vo = ho_v[pl.ds(64 * g + 16 * q, 16)]
            vi = hi_v[pl.ds(64 * g + 16 * q, 16)]
            for e in range(16):
                r = 16 * q + e
                row_v[r, pl.ds(0, 16)] = jnp.full((16,), 1.0,
                                                  jnp.float32) * vo[e]
                row_v[r, pl.ds(64, 16)] = jnp.full((16,), 1.0,
                                                   jnp.float32) * vi[e]

        pltpu.sync_copy(row_v,
                        out_hbm.at[c, pl.ds(s * _SPT + 64 * g, 64)])


def _sc_degrees(idx3):
    f = pl.kernel(
        _deg_body,
        out_type=jax.ShapeDtypeStruct((_NC, _NHIST, _D), jnp.float32),
        mesh=_mesh,
        compiler_params=_cp,
        scratch_types=[
            pltpu.VMEM((_EPT,), jnp.int32),
            pltpu.VMEM((_NHIST,), jnp.float32),
            pltpu.VMEM((_NHIST,), jnp.float32),
            pltpu.VMEM((_NS, _SPT), jnp.float32),
            pltpu.VMEM((64, _D), jnp.float32),
            pltpu.VMEM_SHARED((_NS, 2, _NHIST), jnp.float32),
        ],
    )
    return f(idx3)


_HLEN = (64, _NCHUNK - 64)


def _agg_phase(h_hbm, acc_sh, srcb, dstb, buf_v, sem, n):
    def gather(j, b):
        return pltpu.async_copy(h_hbm.at[srcb.at[j]], buf_v.at[b], sem)

    def wait(j, b):
        pltpu.make_async_copy(h_hbm.at[srcb.at[j]], buf_v.at[b], sem).wait()

    def scatter(j, b):
        pltpu.sync_copy(buf_v.at[b], acc_sh.at[dstb.at[j]], add=True)

    gather(0, 0)

    @pl.loop(0, n // 2)
    def _(i):
        j = 2 * i
        wait(j, 0)
        gather(j + 1, 1)
        scatter(j, 0)
        wait(j + 1, 1)

        @pl.when(j + 2 < n)
        def _():
            gather(j + 2, 0)

        scatter(j + 1, 1)

    if n % 2:
        wait(n - 1, 0)
        scatter(n - 1, 0)


def _agg_body(h_hbm, src_hbm, dst_hbm, out_hbm,
              src_v, dst_v, buf_v, sem, acc_sh):
    c = lax.axis_index("c")
    s = lax.axis_index("s")
    wid = s * _NC + c
    _fill_rows(buf_v.at[0], 8, lambda l: 0.0)
    _zero_acc_slice(buf_v.at[0].at[pl.ds(0, 8)], acc_sh, s)
    plsc.subcore_barrier()

    for h, n in enumerate(_HLEN):
        base = h * _HLEN[0]
        pltpu.sync_copy(src_hbm.at[wid, pl.ds(base, n)], src_v.at[pl.ds(0, n)])
        pltpu.sync_copy(dst_hbm.at[wid, pl.ds(base, n)], dst_v.at[pl.ds(0, n)])
        _agg_phase(h_hbm, acc_sh, src_v, dst_v, buf_v, sem, n)

    plsc.subcore_barrier()
    pltpu.sync_copy(acc_sh.at[pl.ds(s * _RPT, _RPT)],
                    out_hbm.at[c, pl.ds(s * _RPT, _RPT)])


def _sc_aggregate(h, src3, dst3):
    f = pl.kernel(
        _agg_body,
        out_type=jax.ShapeDtypeStruct((_NC, _NPAD, _D), jnp.float32),
        mesh=_mesh,
        scratch_types=[
            pltpu.VMEM((_HLEN[0], _CHUNK), jnp.int32),
            pltpu.VMEM((_HLEN[0], _CHUNK), jnp.int32),
            pltpu.VMEM((2, _CHUNK, _D), jnp.float32),
            pltpu.SemaphoreType.DMA,
            pltpu.VMEM_SHARED((_NPAD, _D), jnp.float32),
        ],
    )
    return f(h, src3, dst3)


_BLK = 1000
_NBLK = _N // _BLK


def _norm_from(degp_ref, k):
    deg = degp_ref[0, :, 64 * k] + degp_ref[1, :, 64 * k]
    return lax.rsqrt(jnp.maximum(deg, 1.0))


def _scale_body(feat_ref, degp_ref, o_ref):
    o_ref[...] = feat_ref[...] * _norm_from(degp_ref, 0)[:, None]


def _tc_scale_src(features, degp):
    return pl.pallas_call(
        _scale_body,
        grid=(_NBLK,),
        in_specs=[
            pl.BlockSpec((_BLK, _D), lambda i: (i, 0)),
            pl.BlockSpec((_NC, _BLK, _D), lambda i: (0, i, 0)),
        ],
        out_specs=pl.BlockSpec((_BLK, _D), lambda i: (i, 0)),
        out_shape=jax.ShapeDtypeStruct((_NPAD, _D), jnp.float32),
    )(features, degp)


def _mid_body(aggp_ref, degp_ref, w_ref, b_ref, o_ref):
    agg = aggp_ref[0] + aggp_ref[1]
    agg = agg * _norm_from(degp_ref, 1)[:, None]
    x = lax.dot_general(agg, w_ref[...], (((1,), (0,)), ((), ())),
                        precision=lax.Precision.HIGHEST,
                        preferred_element_type=jnp.float32)
    x = jnp.maximum(x + b_ref[...], 0.0)
    o_ref[...] = x * _norm_from(degp_ref, 0)[:, None]


def _tc_mid(aggp, degp, W, b):
    return pl.pallas_call(
        _mid_body,
        grid=(_NBLK,),
        in_specs=[
            pl.BlockSpec((_NC, _BLK, _D), lambda i: (0, i, 0)),
            pl.BlockSpec((_NC, _BLK, _D), lambda i: (0, i, 0)),
            pl.BlockSpec((_D, _D), lambda i: (0, 0)),
            pl.BlockSpec((1, _D), lambda i: (0, 0)),
        ],
        out_specs=pl.BlockSpec((_BLK, _D), lambda i: (i, 0)),
        out_shape=jax.ShapeDtypeStruct((_NPAD, _D), jnp.float32),
    )(aggp, degp, W, b.reshape(1, _D))


def _head_body(aggp_ref, degp_ref, w_ref, b_ref, wc_ref, bc_ref,
               o_ref, acc_ref):
    i = pl.program_id(0)
    agg = aggp_ref[0] + aggp_ref[1]
    agg = agg * _norm_from(degp_ref, 1)[:, None]
    x = lax.dot_general(agg, w_ref[...], (((1,), (0,)), ((), ())),
                        precision=lax.Precision.HIGHEST,
                        preferred_element_type=jnp.float32)
    x = jnp.maximum(x + b_ref[...], 0.0)
    part = jnp.sum(x, axis=0, keepdims=True)

    @pl.when(i == 0)
    def _():
        acc_ref[...] = jnp.zeros_like(acc_ref)

    acc_ref[0:1, :] += part

    @pl.when(i == _NBLK - 1)
    def _():
        hg = acc_ref[0:1, :] * (1.0 / _N)
        o_ref[...] = lax.dot_general(
            hg, wc_ref[...], (((1,), (0,)), ((), ())),
            precision=lax.Precision.HIGHEST,
            preferred_element_type=jnp.float32) + bc_ref[...]


def _tc_head(aggp, degp, W, b, Wc, bc):
    return pl.pallas_call(
        _head_body,
        grid=(_NBLK,),
        in_specs=[
            pl.BlockSpec((_NC, _BLK, _D), lambda i: (0, i, 0)),
            pl.BlockSpec((_NC, _BLK, _D), lambda i: (0, i, 0)),
            pl.BlockSpec((_D, _D), lambda i: (0, 0)),
            pl.BlockSpec((1, _D), lambda i: (0, 0)),
            pl.BlockSpec((_D, 10), lambda i: (0, 0)),
            pl.BlockSpec((1, 10), lambda i: (0, 0)),
        ],
        out_specs=pl.BlockSpec((1, 10), lambda i: (0, 0)),
        out_shape=jax.ShapeDtypeStruct((1, 10), jnp.float32),
        scratch_shapes=[pltpu.VMEM((8, _D), jnp.float32)],
    )(aggp, degp, W, b.reshape(1, _D), Wc, bc.reshape(1, 10))


def kernel(features, edge_index, W1, b1, W2, b2, Wc, bc):
    idx3 = edge_index.reshape(2, _NW, _EPT)
    idxp = jnp.pad(idx3, ((0, 0), (0, 0), (0, _EPTP - _EPT)),
                   constant_values=_NPAD - 1)
    idx4 = idxp.reshape(2, _NW, _NCHUNK, _CHUNK)
    src3 = idx4[0]
    dst3 = idx4[1]

    degp = _sc_degrees(idx3)
    h1 = _tc_scale_src(features, degp)
    agg1 = _sc_aggregate(h1, src3, dst3)
    h2 = _tc_mid(agg1, degp, W1, b1)
    agg2 = _sc_aggregate(h2, src3, dst3)
    return _tc_head(agg2, degp, W2, b2, Wc, bc)
